# baseline (speedup 1.0000x reference)
import functools

import jax
import jax.numpy as jnp
from jax import lax
from jax.experimental import pallas as pl
from jax.experimental.pallas import tpu as pltpu
from jax.experimental.pallas import tpu_sc as plsc

NC = 2
NS = 16
LK = 128
HW = 16
IDXB = 16

P = 10112
HALF = P // 2
PH = 5072
WR = 320
WL = HALF - (NS - 1) * WR
ZR = 320
ZL = PH - (NS - 1) * ZR
TRL = HALF


def _scatter_body(with_hist, src_hbm, gidx_hbm, sidx_hbm, *refs):
    if with_hist:
        (out_hbm, b_hbm, d_hbm, gidx_v, sidx_v, sidx2_v, rows_v,
         ones1_v, zeros1_v, acc_sh, bacc_sh, dacc_sh, gsem) = refs
    else:
        out_hbm, gidx_v, sidx_v, rows_v, acc_sh, gsem = refs
    f = rows_v.shape[1]
    ch = gidx_hbm.shape[1]
    c = lax.axis_index("c")
    s = lax.axis_index("s")

    half_base = pl.multiple_of(c * HALF, 8)
    cbase = c * HALF
    zvec = jnp.zeros((16,), jnp.float32)
    ovec = jnp.ones((16,), jnp.float32)

    def _fill(i, carry):
        for k in range(f // 16):
            rows_v[i, pl.ds(k * 16, 16)] = zvec
        return carry

    lax.fori_loop(0, LK, _fill, 0)
    if with_hist:
        for k in range(LK // 16):
            ones1_v[pl.ds(k * 16, 16)] = ovec
            zeros1_v[pl.ds(k * 16, 16)] = zvec

    zbase = pl.multiple_of(s * ZR, 8)

    def _zero(nrows):
        def impl():
            off = 0
            while off < nrows:
                nr = min(LK, nrows - off)
                pltpu.sync_copy(rows_v.at[pl.ds(0, nr)],
                                acc_sh.at[pl.ds(zbase + off, nr)])
                if with_hist:
                    pltpu.sync_copy(zeros1_v.at[pl.ds(0, nr)],
                                    bacc_sh.at[pl.ds(zbase + off, nr)])
                    pltpu.sync_copy(zeros1_v.at[pl.ds(0, nr)],
                                    dacc_sh.at[pl.ds(zbase + off, nr)])
                off += nr
        return impl

    pl.when(s < NS - 1)(_zero(ZR))
    pl.when(s == NS - 1)(_zero(ZL))
    plsc.subcore_barrier()

    def _block(b, carry):
        boff = pl.multiple_of(b * IDXB, 8)
        pltpu.sync_copy(gidx_hbm.at[s, pl.ds(boff, IDXB)], gidx_v)
        pltpu.sync_copy(sidx_hbm.at[c, s, pl.ds(boff, IDXB)], sidx_v)
        if with_hist:
            def _loc(i, carry2):
                for k in range(LK // 16):
                    v = gidx_v[i, pl.ds(k * 16, 16)]
                    lv = v - cbase
                    ok = (lv >= 0) & (lv < HALF)
                    sidx2_v[i, pl.ds(k * 16, 16)] = jnp.where(ok, lv, TRL)
                return carry2

            lax.fori_loop(0, IDXB, _loc, 0)

        def _chunk(j, carry2):
            g = gidx_v.at[j]
            t = sidx_v.at[j]
            pltpu.async_copy(src_hbm.at[g], rows_v, gsem).wait()
            pltpu.sync_copy(rows_v, acc_sh.at[t], add=True)
            if with_hist:
                pltpu.sync_copy(ones1_v, bacc_sh.at[t], add=True)
                pltpu.sync_copy(ones1_v, dacc_sh.at[sidx2_v.at[j]], add=True)
            return carry2

        lax.fori_loop(0, IDXB, _chunk, 0)
        return carry

    lax.fori_loop(0, ch // IDXB, _block, 0)
    plsc.subcore_barrier()

    wbase = pl.multiple_of(s * WR, 8)

    def _wout(nrows):
        def impl():
            pltpu.sync_copy(acc_sh.at[pl.ds(wbase, nrows)],
                            out_hbm.at[pl.ds(half_base + wbase, nrows)])
            if with_hist:
                off = 0
                while off < nrows:
                    nr = min(LK, nrows - off)
                    for hacc, hout in ((bacc_sh, b_hbm), (dacc_sh, d_hbm)):
                        pltpu.sync_copy(hacc.at[pl.ds(wbase + off, nr)],
                                        zeros1_v.at[pl.ds(0, nr)])
                        pltpu.sync_copy(
                            zeros1_v.at[pl.ds(0, nr)],
                            hout.at[pl.ds(half_base + wbase + off, nr)])
                    off += nr
        return impl

    pl.when(s < NS - 1)(_wout(WR))
    pl.when(s == NS - 1)(_wout(WL))


@functools.lru_cache(maxsize=None)
def _scatter_fn(F, with_hist, chunks):
    f32 = jnp.float32
    idx = pltpu.VMEM((IDXB, LK), jnp.int32)
    ot = [jax.ShapeDtypeStruct((P, F), f32)]
    if with_hist:
        ot += [jax.ShapeDtypeStruct((P,), f32), jax.ShapeDtypeStruct((P,), f32)]
        st = [idx, idx, idx, pltpu.VMEM((LK, F), f32),
              pltpu.VMEM((LK,), f32), pltpu.VMEM((LK,), f32),
              pltpu.VMEM_SHARED((PH, F), f32),
              pltpu.VMEM_SHARED((PH,), f32), pltpu.VMEM_SHARED((PH,), f32),
              pltpu.SemaphoreType.DMA]

        def body(src, gidx, sidx, out, b, d, gv, sv, s2v, rv, o1v,
                 z1v, a, ba, da, sem):
            _scatter_body(True, src, gidx, sidx, out, b, d, gv, sv, s2v,
                          rv, o1v, z1v, a, ba, da, sem)
    else:
        st = [idx, idx, pltpu.VMEM((LK, F), f32),
              pltpu.VMEM_SHARED((PH, F), f32),
              pltpu.SemaphoreType.DMA]

        def body(src, gidx, sidx, out, gv, sv, rv, a, sem):
            _scatter_body(False, src, gidx, sidx, out, gv, sv, rv, a, sem)
    mesh = plsc.VectorSubcoreMesh(core_axis_name="c", subcore_axis_name="s",
                                  num_cores=NC, num_subcores=NS)
    return pl.kernel(body, out_type=ot, mesh=mesh, scratch_types=st)


def _mm_body(x_ref, w_ref, o_ref):
    o_ref[...] = jnp.dot(x_ref[...], w_ref[...],
                         preferred_element_type=jnp.float32)


def _matmul(x, w, blk=1264):
    p, fi = x.shape
    fo = w.shape[1]
    return pl.pallas_call(
        _mm_body,
        grid=(p // blk,),
        in_specs=[pl.BlockSpec((blk, fi), lambda i: (i, 0)),
                  pl.BlockSpec((fi, fo), lambda i: (0, 0))],
        out_specs=pl.BlockSpec((blk, fo), lambda i: (i, 0)),
        out_shape=jax.ShapeDtypeStruct((p, fo), jnp.float32),
    )(x, w)


def _scale_body(s_ref, h_ref, o_ref):
    t = h_ref[...]
    inv = jnp.where(t > 0, 1.0 / t, 0.0)
    o_ref[...] = s_ref[...] * inv


def _combine_scale(s, hist, blk=1264):
    p, f = s.shape
    return pl.pallas_call(
        _scale_body,
        grid=(p // blk,),
        in_specs=[pl.BlockSpec((blk, f), lambda i: (i, 0)),
                  pl.BlockSpec((blk, 1), lambda i: (i, 0))],
        out_specs=pl.BlockSpec((blk, f), lambda i: (i, 0)),
        out_shape=jax.ShapeDtypeStruct((p, f), jnp.float32),
    )(s, hist)


def _relu_body(s_ref, d_ref, b_ref, o_ref):
    t = d_ref[...]
    inv = jnp.where(t > 0, 1.0 / t, 0.0)
    h = s_ref[...] * inv + b_ref[...][None, :]
    o_ref[...] = jnp.maximum(h, 0.0)


def _relu_scale(s, d, b1, blk=1264):
    p, f = s.shape
    return pl.pallas_call(
        _relu_body,
        grid=(p // blk,),
        in_specs=[pl.BlockSpec((blk, f), lambda i: (i, 0)),
                  pl.BlockSpec((blk, 1), lambda i: (i, 0)),
                  pl.BlockSpec((f,), lambda i: (0,))],
        out_specs=pl.BlockSpec((blk, f), lambda i: (i, 0)),
        out_shape=jax.ShapeDtypeStruct((p, f), jnp.float32),
    )(s, d, b1)


def _final_body(s_ref, d_ref, w_ref, b_ref, o_ref):
    t = d_ref[...]
    inv = jnp.where(t > 0, 1.0 / t, 0.0)
    z = jnp.dot(s_ref[...] * inv, w_ref[...],
                preferred_element_type=jnp.float32) + b_ref[...][None, :]
    m = jnp.max(z, axis=1, keepdims=True)
    e = jnp.exp(z - m)
    lse = jnp.log(jnp.sum(e, axis=1, keepdims=True)) + m
    o_ref[...] = z - lse


def _final(s, d, w2, b2, n_out, blk=400):
    p, f = s.shape
    fo = w2.shape[1]
    return pl.pallas_call(
        _final_body,
        grid=(n_out // blk,),
        in_specs=[pl.BlockSpec((blk, f), lambda i: (i, 0)),
                  pl.BlockSpec((blk, 1), lambda i: (i, 0)),
                  pl.BlockSpec((f, fo), lambda i: (0, 0)),
                  pl.BlockSpec((fo,), lambda i: (0,))],
        out_specs=pl.BlockSpec((blk, fo), lambda i: (i, 0)),
        out_shape=jax.ShapeDtypeStruct((n_out, fo), jnp.float32),
    )(s, d, w2, b2)


def kernel(x, hyperedge_index, W1, b1, W2, b2):
    n = x.shape[0]
    e = hyperedge_index.shape[1]
    trash = n

    blk_pairs = LK * IDXB
    pairs_per_tile = -(-e // (NS * blk_pairs)) * blk_pairs
    chunks = pairs_per_tile // LK
    pe = pairs_per_tile * NS
    pad = jnp.full((pe - e,), trash, jnp.int32)
    node_p = jnp.concatenate([hyperedge_index[0], pad])
    edge_p = jnp.concatenate([hyperedge_index[1], pad])

    def localize(idx):
        lo = jnp.where(idx < HALF, idx, TRL)
        hi = jnp.where(idx >= HALF, idx - HALF, TRL)
        return jnp.stack([lo, hi]).reshape(NC, NS, chunks, LK)

    node_g = node_p.reshape(NS, chunks, LK)
    edge_g = edge_p.reshape(NS, chunks, LK)
    node_l = localize(node_p)
    edge_l = localize(edge_p)

    x_p = jnp.zeros((P, x.shape[1]), jnp.float32).at[:n].set(x)

    xw1 = _matmul(x_p, W1)
    s1, bh, dh = _scatter_fn(128, True, chunks)(xw1, node_g, edge_l)
    bh, dh = bh[:, None], dh[:, None]
    oute1 = _combine_scale(s1, bh)
    (s2,) = _scatter_fn(128, False, chunks)(oute1, edge_g, node_l)
    g = _relu_scale(s2, dh, b1)
    (s1b,) = _scatter_fn(128, False, chunks)(g, node_g, edge_l)
    oute2 = _combine_scale(s1b, bh)
    (s2b,) = _scatter_fn(128, False, chunks)(oute2, edge_g, node_l)
    return _final(s2b, dh, W2, b2, n)
